# Initial kernel scaffold; baseline (speedup 1.0000x reference)
import jax
import jax.numpy as jnp
from jax.experimental import pallas as pl


def kernel(x_vals, x_cons, edge_index, params):
    raise NotImplementedError("write your pallas kernel here")


# speedup vs baseline: 3.8820x; 3.8820x over previous
import functools

import jax
import jax.numpy as jnp
from jax import lax
from jax.experimental import pallas as pl
from jax.experimental.pallas import tpu as pltpu
from jax.experimental.pallas import tpu_sc as plsc

HID = 64
N_NODES = 50000
N_EDGES = 800000
HALF = 25000
TILE_ROWS = 1568
ACC_ROWS = 16 * TILE_ROWS
E_PER_TILE = N_EDGES // 16
SUPER = 2000
CHUNK = 80
NSUP = E_PER_TILE // SUPER
NCH = SUPER // CHUNK


def _encode_body(x_ref, w1_ref, b1_ref, w2_ref, b2_ref, o_ref):
    h = jnp.dot(x_ref[...], w1_ref[...], preferred_element_type=jnp.float32)
    h = jnp.maximum(h + b1_ref[...], 0.0)
    h = jnp.dot(h, w2_ref[...], preferred_element_type=jnp.float32)
    o_ref[...] = jnp.maximum(h + b2_ref[...], 0.0)


def _encode(x, p1, p2):
    n, din = x.shape
    blk = 2000
    return pl.pallas_call(
        _encode_body,
        grid=(n // blk,),
        in_specs=[
            pl.BlockSpec((blk, din), lambda i: (i, 0)),
            pl.BlockSpec((din, HID), lambda i: (0, 0)),
            pl.BlockSpec((1, HID), lambda i: (0, 0)),
            pl.BlockSpec((HID, HID), lambda i: (0, 0)),
            pl.BlockSpec((1, HID), lambda i: (0, 0)),
        ],
        out_specs=pl.BlockSpec((blk, HID), lambda i: (i, 0)),
        out_shape=jax.ShapeDtypeStruct((n, HID), jnp.float32),
    )(x, p1["W"], p1["b"].reshape(1, HID), p2["W"], p2["b"].reshape(1, HID))


def _update_body(m_ref, h_ref, wm_ref, wh_ref, b_ref, o_ref):
    z = (jnp.dot(m_ref[...], wm_ref[...], preferred_element_type=jnp.float32)
         + jnp.dot(h_ref[...], wh_ref[...], preferred_element_type=jnp.float32)
         + b_ref[...])
    mu = jnp.mean(z, axis=-1, keepdims=True)
    zc = z - mu
    var = jnp.mean(zc * zc, axis=-1, keepdims=True)
    o_ref[...] = jnp.maximum(zc * lax.rsqrt(var + 1e-5), 0.0)


def _update(msg, h, wm, wh, b):
    n = h.shape[0]
    blk = 2000
    return pl.pallas_call(
        _update_body,
        grid=(n // blk,),
        in_specs=[
            pl.BlockSpec((blk, HID), lambda i: (i, 0)),
            pl.BlockSpec((blk, HID), lambda i: (i, 0)),
            pl.BlockSpec((HID, HID), lambda i: (0, 0)),
            pl.BlockSpec((HID, HID), lambda i: (0, 0)),
            pl.BlockSpec((1, HID), lambda i: (0, 0)),
        ],
        out_specs=pl.BlockSpec((blk, HID), lambda i: (i, 0)),
        out_shape=jax.ShapeDtypeStruct((n, HID), jnp.float32),
    )(msg, h, wm, wh, b.reshape(1, HID))


def _pred_body(h_ref, wp_ref, bp_ref, wo_ref, bo_ref, o_ref):
    h = jnp.dot(h_ref[...], wp_ref[...], preferred_element_type=jnp.float32)
    h = jnp.maximum(h + bp_ref[...], 0.0)
    o_ref[...] = jnp.sum(h * wo_ref[...], axis=1) + bo_ref[0, 0]


def _pred(h, pred_p, out_p):
    n = h.shape[0]
    blk = 2048
    grid = (n + blk - 1) // blk
    out = pl.pallas_call(
        _pred_body,
        grid=(grid,),
        in_specs=[
            pl.BlockSpec((blk, HID), lambda i: (i, 0)),
            pl.BlockSpec((HID, HID), lambda i: (0, 0)),
            pl.BlockSpec((1, HID), lambda i: (0, 0)),
            pl.BlockSpec((1, HID), lambda i: (0, 0)),
            pl.BlockSpec((1, 1), lambda i: (0, 0)),
        ],
        out_specs=pl.BlockSpec((blk,), lambda i: (i,)),
        out_shape=jax.ShapeDtypeStruct((grid * blk,), jnp.float32),
    )(h, pred_p["W"], pred_p["b"].reshape(1, HID),
      out_p["W"].reshape(1, HID), out_p["b"].reshape(1, 1))
    return out[:n]


def _segsum_body(table, gidx, sidx, zrows, out, acc, gsb, ssb, dbuf, rows, sem):
    c = lax.axis_index("c")
    s = lax.axis_index("s")
    tile_base = s * TILE_ROWS
    pltpu.sync_copy(zrows, acc.at[pl.ds(tile_base, TILE_ROWS)])
    plsc.subcore_barrier()

    half_base = c * HALF
    ebase = s * E_PER_TILE

    def sup_body(j, carry):
        sb = ebase + j * SUPER
        pltpu.sync_copy(gidx.at[pl.ds(sb, SUPER)], gsb)
        pltpu.sync_copy(sidx.at[pl.ds(sb, SUPER)], ssb)

        def ch_body(q, carry2):
            off = q * CHUNK
            cp = pltpu.async_copy(table.at[gsb.at[pl.ds(off, CHUNK)]],
                                  rows, sem)
            for t in range(CHUNK // 16):
                g = ssb[pl.ds(off + t * 16, 16)]
                loc = g - half_base
                ok = (loc >= 0) & (loc < HALF)
                dbuf[pl.ds(t * 16, 16)] = jnp.where(ok, loc, HALF)
            cp.wait()
            pltpu.sync_copy(rows, acc.at[dbuf], add=True)
            return carry2

        return lax.fori_loop(0, NCH, ch_body, carry)

    lax.fori_loop(0, NSUP, sup_body, 0)
    plsc.subcore_barrier()
    pltpu.sync_copy(acc.at[pl.ds(tile_base, TILE_ROWS)],
                    out.at[pl.ds(c * ACC_ROWS + tile_base, TILE_ROWS)])


@functools.cache
def _segsum_call():
    return pl.kernel(
        _segsum_body,
        out_type=jax.ShapeDtypeStruct((2 * ACC_ROWS, HID), jnp.float32),
        mesh=plsc.VectorSubcoreMesh(core_axis_name="c", subcore_axis_name="s",
                                    num_cores=2, num_subcores=16),
        scratch_types=[
            pltpu.VMEM_SHARED((ACC_ROWS, HID), jnp.float32),
            pltpu.VMEM((SUPER,), jnp.int32),
            pltpu.VMEM((SUPER,), jnp.int32),
            pltpu.VMEM((CHUNK,), jnp.int32),
            pltpu.VMEM((CHUNK, HID), jnp.float32),
            pltpu.SemaphoreType.DMA,
        ],
        compiler_params=pltpu.CompilerParams(use_tc_tiling_on_sc=False),
    )


def _segment_sum(table, g_idx, s_idx, zrows):
    out = _segsum_call()(table, g_idx, s_idx, zrows)
    return out.reshape(2, ACC_ROWS, HID)[:, :HALF].reshape(N_NODES, HID)


def kernel(x_vals, x_cons, edge_index, params):
    hv = _encode(x_vals, *params["enc_v"])
    hc = _encode(x_cons, *params["enc_c"])
    row = edge_index[0].astype(jnp.int32)
    col = edge_index[1].astype(jnp.int32)
    zrows = jnp.zeros((TILE_ROWS, HID), jnp.float32)
    for layer in params["convs"]:
        msg_c = _segment_sum(hv, col, row, zrows)
        hc = _update(msg_c, hc, layer["Wv2c"], layer["Wcs"], layer["bc"])
        msg_v = _segment_sum(hc, row, col, zrows)
        hv = _update(msg_v, hv, layer["Wc2v"], layer["Wvs"], layer["bv"])
    return _pred(hv, params["pred"][0], params["out"])


# speedup vs baseline: 4.9746x; 1.2815x over previous
import functools

import jax
import jax.numpy as jnp
from jax import lax
from jax.experimental import pallas as pl
from jax.experimental.pallas import tpu as pltpu
from jax.experimental.pallas import tpu_sc as plsc

HID = 64
N_NODES = 50000
N_EDGES = 800000
HALF = 25000
TILE_ROWS = 1568
ACC_ROWS = 16 * TILE_ROWS
E_PER_TILE = N_EDGES // 16
SUPER = 2000
CHUNK = 80
NSUP = E_PER_TILE // SUPER
NCH = SUPER // CHUNK


def _encode_body(x_ref, w1_ref, b1_ref, w2_ref, b2_ref, o_ref):
    h = jnp.dot(x_ref[...], w1_ref[...], preferred_element_type=jnp.float32)
    h = jnp.maximum(h + b1_ref[...], 0.0)
    h = jnp.dot(h, w2_ref[...], preferred_element_type=jnp.float32)
    o_ref[...] = jnp.maximum(h + b2_ref[...], 0.0)


def _encode(x, p1, p2):
    n, din = x.shape
    blk = 2000
    return pl.pallas_call(
        _encode_body,
        grid=(n // blk,),
        in_specs=[
            pl.BlockSpec((blk, din), lambda i: (i, 0)),
            pl.BlockSpec((din, HID), lambda i: (0, 0)),
            pl.BlockSpec((1, HID), lambda i: (0, 0)),
            pl.BlockSpec((HID, HID), lambda i: (0, 0)),
            pl.BlockSpec((1, HID), lambda i: (0, 0)),
        ],
        out_specs=pl.BlockSpec((blk, HID), lambda i: (i, 0)),
        out_shape=jax.ShapeDtypeStruct((n, HID), jnp.float32),
    )(x, p1["W"], p1["b"].reshape(1, HID), p2["W"], p2["b"].reshape(1, HID))


def _update_body(m_ref, h_ref, wm_ref, wh_ref, b_ref, o_ref):
    z = (jnp.dot(m_ref[...], wm_ref[...], preferred_element_type=jnp.float32)
         + jnp.dot(h_ref[...], wh_ref[...], preferred_element_type=jnp.float32)
         + b_ref[...])
    mu = jnp.mean(z, axis=-1, keepdims=True)
    zc = z - mu
    var = jnp.mean(zc * zc, axis=-1, keepdims=True)
    o_ref[...] = jnp.maximum(zc * lax.rsqrt(var + 1e-5), 0.0)


def _update(msg, h, wm, wh, b):
    n = h.shape[0]
    blk = 2000
    return pl.pallas_call(
        _update_body,
        grid=(n // blk,),
        in_specs=[
            pl.BlockSpec((blk, HID), lambda i: (i, 0)),
            pl.BlockSpec((blk, HID), lambda i: (i, 0)),
            pl.BlockSpec((HID, HID), lambda i: (0, 0)),
            pl.BlockSpec((HID, HID), lambda i: (0, 0)),
            pl.BlockSpec((1, HID), lambda i: (0, 0)),
        ],
        out_specs=pl.BlockSpec((blk, HID), lambda i: (i, 0)),
        out_shape=jax.ShapeDtypeStruct((n, HID), jnp.float32),
    )(msg, h, wm, wh, b.reshape(1, HID))


def _pred_body(h_ref, wp_ref, bp_ref, wo_ref, bo_ref, o_ref):
    h = jnp.dot(h_ref[...], wp_ref[...], preferred_element_type=jnp.float32)
    h = jnp.maximum(h + bp_ref[...], 0.0)
    o_ref[...] = jnp.sum(h * wo_ref[...], axis=1) + bo_ref[0, 0]


def _pred(h, pred_p, out_p):
    n = h.shape[0]
    blk = 2048
    grid = (n + blk - 1) // blk
    out = pl.pallas_call(
        _pred_body,
        grid=(grid,),
        in_specs=[
            pl.BlockSpec((blk, HID), lambda i: (i, 0)),
            pl.BlockSpec((HID, HID), lambda i: (0, 0)),
            pl.BlockSpec((1, HID), lambda i: (0, 0)),
            pl.BlockSpec((1, HID), lambda i: (0, 0)),
            pl.BlockSpec((1, 1), lambda i: (0, 0)),
        ],
        out_specs=pl.BlockSpec((blk,), lambda i: (i,)),
        out_shape=jax.ShapeDtypeStruct((grid * blk,), jnp.float32),
    )(h, pred_p["W"], pred_p["b"].reshape(1, HID),
      out_p["W"].reshape(1, HID), out_p["b"].reshape(1, 1))
    return out[:n]


NBUF = 5


def _segsum_body(table, gidx, sidx, zrows, out, acc, gsb, ssb, dbuf, rows,
                 gsem):
    c = lax.axis_index("c")
    s = lax.axis_index("s")
    tile_base = s * TILE_ROWS
    pltpu.sync_copy(zrows, acc.at[pl.ds(tile_base, TILE_ROWS)])
    plsc.subcore_barrier()

    half_base = c * HALF
    ebase = s * E_PER_TILE

    def sup_body(j, carry):
        sb = ebase + j * SUPER
        pltpu.sync_copy(gidx.at[pl.ds(sb, SUPER)], gsb)
        pltpu.sync_copy(sidx.at[pl.ds(sb, SUPER)], ssb)

        for b in range(NBUF):
            pltpu.async_copy(table.at[gsb.at[pl.ds(b * CHUNK, CHUNK)]],
                             rows.at[b], gsem.at[b])

        def ch_body(qq, carry2):
            for b in range(NBUF):
                q = qq * NBUF + b
                off = q * CHUNK
                pltpu.make_async_copy(
                    table.at[gsb.at[pl.ds(off, CHUNK)]],
                    rows.at[b], gsem.at[b]).wait()
                for t in range(CHUNK // 16):
                    g = ssb[pl.ds(off + t * 16, 16)]
                    loc = g - half_base
                    ok = (loc >= 0) & (loc < HALF)
                    dbuf[pl.ds(t * 16, 16)] = jnp.where(ok, loc, HALF)
                pltpu.sync_copy(rows.at[b], acc.at[dbuf], add=True)

                @pl.when(qq < NCH // NBUF - 1)
                def _():
                    off2 = off + NBUF * CHUNK
                    pltpu.async_copy(
                        table.at[gsb.at[pl.ds(off2, CHUNK)]],
                        rows.at[b], gsem.at[b])
            return carry2

        return lax.fori_loop(0, NCH // NBUF, ch_body, carry)

    lax.fori_loop(0, NSUP, sup_body, 0)
    plsc.subcore_barrier()
    pltpu.sync_copy(acc.at[pl.ds(tile_base, TILE_ROWS)],
                    out.at[pl.ds(c * ACC_ROWS + tile_base, TILE_ROWS)])


@functools.cache
def _segsum_call():
    return pl.kernel(
        _segsum_body,
        out_type=jax.ShapeDtypeStruct((2 * ACC_ROWS, HID), jnp.float32),
        mesh=plsc.VectorSubcoreMesh(core_axis_name="c", subcore_axis_name="s",
                                    num_cores=2, num_subcores=16),
        scratch_types=[
            pltpu.VMEM_SHARED((ACC_ROWS, HID), jnp.float32),
            pltpu.VMEM((SUPER,), jnp.int32),
            pltpu.VMEM((SUPER,), jnp.int32),
            pltpu.VMEM((CHUNK,), jnp.int32),
            pltpu.VMEM((NBUF, CHUNK, HID), jnp.float32),
            pltpu.SemaphoreType.DMA((NBUF,)),
        ],
        compiler_params=pltpu.CompilerParams(use_tc_tiling_on_sc=False),
    )


def _segment_sum(table, g_idx, s_idx, zrows):
    out = _segsum_call()(table, g_idx, s_idx, zrows)
    return out.reshape(2, ACC_ROWS, HID)[:, :HALF].reshape(N_NODES, HID)


def kernel(x_vals, x_cons, edge_index, params):
    hv = _encode(x_vals, *params["enc_v"])
    hc = _encode(x_cons, *params["enc_c"])
    row = edge_index[0].astype(jnp.int32)
    col = edge_index[1].astype(jnp.int32)
    zrows = jnp.zeros((TILE_ROWS, HID), jnp.float32)
    for layer in params["convs"]:
        msg_c = _segment_sum(hv, col, row, zrows)
        hc = _update(msg_c, hc, layer["Wv2c"], layer["Wcs"], layer["bc"])
        msg_v = _segment_sum(hc, row, col, zrows)
        hv = _update(msg_v, hv, layer["Wc2v"], layer["Wvs"], layer["bv"])
    return _pred(hv, params["pred"][0], params["out"])
